# baseline (speedup 1.0000x reference)
import functools

import numpy as np
import jax
import jax.numpy as jnp
from jax import lax
from jax.experimental import pallas as pl
from jax.experimental.pallas import tpu as pltpu
from jax.experimental.pallas import tpu_sc as plsc

BETA = 1.0
GAMMA = 1.0
DELTA = 0.0
INTERPOLATE_N = 4
LAMBD = 0.05


def _interp_weights(n=INTERPOLATE_N):
    t = np.linspace(0.0, 1.0, n)
    a, b = np.meshgrid(t, t, indexing="ij")
    w = np.stack([a, b, 1.0 - a - b], axis=-1).reshape(-1, 3)
    return np.asarray(w[w[:, 2] >= 0], dtype=np.float32)


_W = _interp_weights()
_K = _W.shape[0]


@functools.lru_cache(maxsize=None)
def _make_sampler(B, M, F):
    info = plsc.get_sparse_core_info()
    NC, NS = info.num_cores, info.num_subcores
    NW = NC * NS
    WPB = NW // B
    FPW = F // WPB
    G = FPW // 16
    SPW = FPW * _K

    mesh = plsc.VectorSubcoreMesh(core_axis_name="c", subcore_axis_name="s")

    S = F * _K

    @functools.partial(
        pl.kernel,
        mesh=mesh,
        compiler_params=pltpu.CompilerParams(needs_layout_passes=False),
        out_type=jax.ShapeDtypeStruct((B * 3 * S,), jnp.float32),
        scratch_types=[
            pltpu.VMEM((B * M * 3,), jnp.float32),
            pltpu.VMEM((FPW * 3,), jnp.int32),
            pltpu.VMEM((3 * SPW,), jnp.float32),
            pltpu.SemaphoreType.DMA,
        ],
    )
    def sampler(cv_hbm, cf_hbm, out_hbm, cv_v, cf_v, out_v, sem):
        wid = lax.axis_index("s") * NC + lax.axis_index("c")
        b = wid // WPB
        wb = wid % WPB
        cf_off = pl.multiple_of(b * (F * 3) + wb * (FPW * 3), 8)
        in_copies = [
            pltpu.async_copy(cv_hbm, cv_v, sem),
            pltpu.async_copy(cf_hbm.at[pl.ds(cf_off, FPW * 3)], cf_v, sem),
        ]
        for cp in in_copies:
            cp.wait()
        cv_base = b * (M * 3)
        iota = lax.broadcasted_iota(jnp.int32, (16,), 0)
        for g in range(G):
            verts = []
            for j in range(3):
                idx = plsc.load_gather(cf_v, [iota * 3 + (g * 48 + j)])
                verts.append(
                    [
                        plsc.load_gather(cv_v, [idx * 3 + (cv_base + c)])
                        for c in range(3)
                    ]
                )
            for k in range(_K):
                for c in range(3):
                    val = None
                    for j in range(3):
                        wkj = float(_W[k, j])
                        if wkj == 0.0:
                            continue
                        term = verts[j][c] if wkj == 1.0 else wkj * verts[j][c]
                        val = term if val is None else val + term
                    out_v[pl.ds(c * SPW + k * FPW + g * 16, 16)] = val
        copies = [
            pltpu.async_copy(
                out_v.at[pl.ds(c * SPW, SPW)],
                out_hbm.at[
                    pl.ds(
                        pl.multiple_of(b * (3 * S) + c * S + wb * SPW, 8), SPW
                    )
                ],
                sem,
            )
            for c in range(3)
        ]
        for cp in copies:
            cp.wait()

    return sampler


@functools.lru_cache(maxsize=None)
def _make_chamfer(B, M, S, N, s_blk):
    nsb = S // s_blk
    gamma_eff = float(GAMMA + DELTA * M)

    def body(cs_ref, sh_ref, loss_ref, s2c_ref, acc_ref):
        b = pl.program_id(0)
        s = pl.program_id(1)

        @pl.when(s == 0)
        def _init():
            s2c_ref[...] = jnp.full_like(s2c_ref[...], jnp.inf)
            acc_ref[0, 0] = 0.0
            acc_ref[1, 0] = -jnp.inf

        csx = cs_ref[0, 0, :][None, :]
        csy = cs_ref[0, 1, :][None, :]
        csz = cs_ref[0, 2, :][None, :]
        cs2 = csx * csx + csy * csy + csz * csz
        shx = sh_ref[0, :, 0:1]
        shy = sh_ref[0, :, 1:2]
        shz = sh_ref[0, :, 2:3]
        sh2 = shx * shx + shy * shy + shz * shz
        lhs = jnp.concatenate(
            [-2.0 * shx, -2.0 * shy, -2.0 * shz, jnp.ones_like(shx), sh2],
            axis=1,
        )
        rhs = jnp.concatenate(
            [csx, csy, csz, cs2, jnp.ones_like(csx)], axis=0
        )
        lh = lhs.astype(jnp.bfloat16)
        ll = (lhs - lh.astype(jnp.float32)).astype(jnp.bfloat16)
        rh = rhs.astype(jnp.bfloat16)
        rl = (rhs - rh.astype(jnp.float32)).astype(jnp.bfloat16)
        d = jnp.dot(
            jnp.concatenate([lh, lh, ll], axis=1),
            jnp.concatenate([rh, rl, rh], axis=0),
            preferred_element_type=jnp.float32,
        )

        c2s = jnp.min(d, axis=0)
        c2s = jnp.where(c2s > LAMBD, c2s, 0.0)
        acc_ref[0, 0] += jnp.sum(c2s)
        acc_ref[1, 0] = jnp.maximum(acc_ref[1, 0], jnp.max(c2s))
        m = d[:, 0:128]
        for i in range(1, s_blk // 128):
            m = jnp.minimum(m, d[:, i * 128 : (i + 1) * 128])
        s2c_ref[...] = jnp.minimum(s2c_ref[...], m)

        @pl.when(s == nsb - 1)
        def _finalize():
            s2c = jnp.min(s2c_ref[...], axis=1, keepdims=True)
            s2c = jnp.where(s2c > LAMBD, s2c, 0.0)
            loss_b = (
                jnp.mean(s2c) * gamma_eff
                + acc_ref[0, 0] / S
                + BETA * acc_ref[1, 0]
            )

            @pl.when(b == 0)
            def _():
                loss_ref[0, 0] = loss_b / B

            @pl.when(b != 0)
            def _():
                loss_ref[0, 0] += loss_b / B

    return pl.pallas_call(
        body,
        grid=(B, nsb),
        in_specs=[
            pl.BlockSpec((1, 3, s_blk), lambda b, s: (b, 0, s)),
            pl.BlockSpec((1, N, 3), lambda b, s: (b, 0, 0)),
        ],
        out_specs=pl.BlockSpec(memory_space=pltpu.SMEM),
        out_shape=jax.ShapeDtypeStruct((1, 1), jnp.float32),
        scratch_shapes=[
            pltpu.VMEM((N, 128), jnp.float32),
            pltpu.SMEM((2, 1), jnp.float32),
        ],
    )


def kernel(cage_v, cage_f, shape):
    B, M, _ = cage_v.shape
    F = cage_f.shape[1]
    N = shape.shape[1]
    S = F * _K

    cv_flat = cage_v.reshape(-1)
    cf_flat = cage_f.reshape(-1)

    cs = _make_sampler(B, M, F)(cv_flat, cf_flat).reshape(B, 3, S)
    loss = _make_chamfer(B, M, S, N, 5120)(cs, shape)
    return loss[0, 0]


# speedup vs baseline: 1.0403x; 1.0403x over previous
import functools

import numpy as np
import jax
import jax.numpy as jnp
from jax import lax
from jax.experimental import pallas as pl
from jax.experimental.pallas import tpu as pltpu
from jax.experimental.pallas import tpu_sc as plsc

BETA = 1.0
GAMMA = 1.0
DELTA = 0.0
INTERPOLATE_N = 4
LAMBD = 0.05


def _interp_weights(n=INTERPOLATE_N):
    t = np.linspace(0.0, 1.0, n)
    a, b = np.meshgrid(t, t, indexing="ij")
    w = np.stack([a, b, 1.0 - a - b], axis=-1).reshape(-1, 3)
    return np.asarray(w[w[:, 2] >= 0], dtype=np.float32)


_W = _interp_weights()
_K = _W.shape[0]


@functools.lru_cache(maxsize=None)
def _make_sampler(B, M, F):
    info = plsc.get_sparse_core_info()
    NC, NS = 1, info.num_subcores
    NW = NC * NS
    WPB = NW // B
    FPW = F // WPB
    G = FPW // 16
    SPW = FPW * _K

    mesh = plsc.VectorSubcoreMesh(
        core_axis_name="c", subcore_axis_name="s", num_cores=NC
    )

    S = F * _K

    @functools.partial(
        pl.kernel,
        mesh=mesh,
        compiler_params=pltpu.CompilerParams(needs_layout_passes=False),
        out_type=jax.ShapeDtypeStruct((B * 3 * S,), jnp.float32),
        scratch_types=[
            pltpu.VMEM((B * M * 3,), jnp.float32),
            pltpu.VMEM((FPW * 3,), jnp.int32),
            pltpu.VMEM((3 * SPW,), jnp.float32),
            pltpu.SemaphoreType.DMA,
        ],
    )
    def sampler(cv_hbm, cf_hbm, out_hbm, cv_v, cf_v, out_v, sem):
        wid = lax.axis_index("s") * NC + lax.axis_index("c")
        b = wid // WPB
        wb = wid % WPB
        cf_off = pl.multiple_of(b * (F * 3) + wb * (FPW * 3), 8)
        in_copies = [
            pltpu.async_copy(cv_hbm, cv_v, sem),
            pltpu.async_copy(cf_hbm.at[pl.ds(cf_off, FPW * 3)], cf_v, sem),
        ]
        for cp in in_copies:
            cp.wait()
        cv_base = b * (M * 3)
        iota = lax.broadcasted_iota(jnp.int32, (16,), 0)
        for g in range(G):
            verts = []
            for j in range(3):
                idx = plsc.load_gather(cf_v, [iota * 3 + (g * 48 + j)])
                verts.append(
                    [
                        plsc.load_gather(cv_v, [idx * 3 + (cv_base + c)])
                        for c in range(3)
                    ]
                )
            for k in range(_K):
                for c in range(3):
                    val = None
                    for j in range(3):
                        wkj = float(_W[k, j])
                        if wkj == 0.0:
                            continue
                        term = verts[j][c] if wkj == 1.0 else wkj * verts[j][c]
                        val = term if val is None else val + term
                    out_v[pl.ds(c * SPW + k * FPW + g * 16, 16)] = val
        copies = [
            pltpu.async_copy(
                out_v.at[pl.ds(c * SPW, SPW)],
                out_hbm.at[
                    pl.ds(
                        pl.multiple_of(b * (3 * S) + c * S + wb * SPW, 8), SPW
                    )
                ],
                sem,
            )
            for c in range(3)
        ]
        for cp in copies:
            cp.wait()

    return sampler


@functools.lru_cache(maxsize=None)
def _make_chamfer(B, M, S, N, s_blk):
    nsb = S // s_blk
    gamma_eff = float(GAMMA + DELTA * M)

    def body(cs_ref, sh_ref, loss_ref, s2c_ref, acc_ref):
        b = pl.program_id(0)
        s = pl.program_id(1)

        @pl.when(s == 0)
        def _init():
            s2c_ref[...] = jnp.full_like(s2c_ref[...], jnp.inf)
            acc_ref[0, 0] = 0.0
            acc_ref[1, 0] = -jnp.inf

        csx = cs_ref[0, 0, :][None, :]
        csy = cs_ref[0, 1, :][None, :]
        csz = cs_ref[0, 2, :][None, :]
        cs2 = csx * csx + csy * csy + csz * csz
        shx = sh_ref[0, :, 0:1]
        shy = sh_ref[0, :, 1:2]
        shz = sh_ref[0, :, 2:3]
        sh2 = shx * shx + shy * shy + shz * shz
        lhs = jnp.concatenate(
            [-2.0 * shx, -2.0 * shy, -2.0 * shz, jnp.ones_like(shx), sh2],
            axis=1,
        )
        rhs = jnp.concatenate(
            [csx, csy, csz, cs2, jnp.ones_like(csx)], axis=0
        )
        lh = lhs.astype(jnp.bfloat16)
        ll = (lhs - lh.astype(jnp.float32)).astype(jnp.bfloat16)
        rh = rhs.astype(jnp.bfloat16)
        rl = (rhs - rh.astype(jnp.float32)).astype(jnp.bfloat16)
        d = jnp.dot(
            jnp.concatenate([lh, lh, ll], axis=1),
            jnp.concatenate([rh, rl, rh], axis=0),
            preferred_element_type=jnp.float32,
        )

        c2s = jnp.min(d, axis=0)
        c2s = jnp.where(c2s > LAMBD, c2s, 0.0)
        acc_ref[0, 0] += jnp.sum(c2s)
        acc_ref[1, 0] = jnp.maximum(acc_ref[1, 0], jnp.max(c2s))
        m = d[:, 0:128]
        for i in range(1, s_blk // 128):
            m = jnp.minimum(m, d[:, i * 128 : (i + 1) * 128])
        s2c_ref[...] = jnp.minimum(s2c_ref[...], m)

        @pl.when(s == nsb - 1)
        def _finalize():
            s2c = jnp.min(s2c_ref[...], axis=1, keepdims=True)
            s2c = jnp.where(s2c > LAMBD, s2c, 0.0)
            loss_b = (
                jnp.mean(s2c) * gamma_eff
                + acc_ref[0, 0] / S
                + BETA * acc_ref[1, 0]
            )

            @pl.when(b == 0)
            def _():
                loss_ref[0, 0] = loss_b / B

            @pl.when(b != 0)
            def _():
                loss_ref[0, 0] += loss_b / B

    return pl.pallas_call(
        body,
        grid=(B, nsb),
        in_specs=[
            pl.BlockSpec((1, 3, s_blk), lambda b, s: (b, 0, s)),
            pl.BlockSpec((1, N, 3), lambda b, s: (b, 0, 0)),
        ],
        out_specs=pl.BlockSpec(memory_space=pltpu.SMEM),
        out_shape=jax.ShapeDtypeStruct((1, 1), jnp.float32),
        scratch_shapes=[
            pltpu.VMEM((N, 128), jnp.float32),
            pltpu.SMEM((2, 1), jnp.float32),
        ],
    )


def kernel(cage_v, cage_f, shape):
    B, M, _ = cage_v.shape
    F = cage_f.shape[1]
    N = shape.shape[1]
    S = F * _K

    cv_flat = cage_v.reshape(-1)
    cf_flat = cage_f.reshape(-1)

    cs = _make_sampler(B, M, F)(cv_flat, cf_flat).reshape(B, 3, S)
    loss = _make_chamfer(B, M, S, N, 5120)(cs, shape)
    return loss[0, 0]


# speedup vs baseline: 1.1007x; 1.0580x over previous
import functools

import numpy as np
import jax
import jax.numpy as jnp
from jax import lax
from jax.experimental import pallas as pl
from jax.experimental.pallas import tpu as pltpu
from jax.experimental.pallas import tpu_sc as plsc

BETA = 1.0
GAMMA = 1.0
DELTA = 0.0
INTERPOLATE_N = 4
LAMBD = 0.05


def _interp_weights(n=INTERPOLATE_N):
    t = np.linspace(0.0, 1.0, n)
    a, b = np.meshgrid(t, t, indexing="ij")
    w = np.stack([a, b, 1.0 - a - b], axis=-1).reshape(-1, 3)
    return np.asarray(w[w[:, 2] >= 0], dtype=np.float32)


_W = _interp_weights()
_K = _W.shape[0]


@functools.lru_cache(maxsize=None)
def _make_sampler(B, M, F):
    info = plsc.get_sparse_core_info()
    NC, NS = 1, info.num_subcores
    NW = NC * NS
    WPB = NW // B
    FPW = F // WPB
    G = FPW // 16
    SPW = FPW * _K

    mesh = plsc.VectorSubcoreMesh(
        core_axis_name="c", subcore_axis_name="s", num_cores=NC
    )

    S = F * _K

    @functools.partial(
        pl.kernel,
        mesh=mesh,
        compiler_params=pltpu.CompilerParams(
            needs_layout_passes=False, skip_device_barrier=True
        ),
        out_type=jax.ShapeDtypeStruct((B * 3 * S,), jnp.float32),
        scratch_types=[
            pltpu.VMEM((B * M * 3,), jnp.float32),
            pltpu.VMEM((FPW * 3,), jnp.int32),
            pltpu.VMEM((3 * SPW,), jnp.float32),
            pltpu.SemaphoreType.DMA,
        ],
    )
    def sampler(cv_hbm, cf_hbm, out_hbm, cv_v, cf_v, out_v, sem):
        wid = lax.axis_index("s") * NC + lax.axis_index("c")
        b = wid // WPB
        wb = wid % WPB
        cf_off = pl.multiple_of(b * (F * 3) + wb * (FPW * 3), 8)
        in_copies = [
            pltpu.async_copy(cv_hbm, cv_v, sem),
            pltpu.async_copy(cf_hbm.at[pl.ds(cf_off, FPW * 3)], cf_v, sem),
        ]
        for cp in in_copies:
            cp.wait()
        cv_base = b * (M * 3)
        iota = lax.broadcasted_iota(jnp.int32, (16,), 0)
        for g in range(G):
            verts = []
            for j in range(3):
                idx = plsc.load_gather(cf_v, [iota * 3 + (g * 48 + j)])
                verts.append(
                    [
                        plsc.load_gather(cv_v, [idx * 3 + (cv_base + c)])
                        for c in range(3)
                    ]
                )
            for k in range(_K):
                for c in range(3):
                    val = None
                    for j in range(3):
                        wkj = float(_W[k, j])
                        if wkj == 0.0:
                            continue
                        term = verts[j][c] if wkj == 1.0 else wkj * verts[j][c]
                        val = term if val is None else val + term
                    out_v[pl.ds(c * SPW + k * FPW + g * 16, 16)] = val
        copies = [
            pltpu.async_copy(
                out_v.at[pl.ds(c * SPW, SPW)],
                out_hbm.at[
                    pl.ds(
                        pl.multiple_of(b * (3 * S) + c * S + wb * SPW, 8), SPW
                    )
                ],
                sem,
            )
            for c in range(3)
        ]
        for cp in copies:
            cp.wait()

    return sampler


@functools.lru_cache(maxsize=None)
def _make_chamfer(B, M, S, N, s_blk):
    nsb = S // s_blk
    gamma_eff = float(GAMMA + DELTA * M)

    def body(cs_ref, sh_ref, loss_ref, s2c_ref, acc_ref):
        b = pl.program_id(0)
        s = pl.program_id(1)

        @pl.when(s == 0)
        def _init():
            s2c_ref[...] = jnp.full_like(s2c_ref[...], jnp.inf)
            acc_ref[0, 0] = 0.0
            acc_ref[1, 0] = -jnp.inf

        csx = cs_ref[pl.ds(0 * S + s * s_blk, s_blk)][None, :]
        csy = cs_ref[pl.ds(1 * S + s * s_blk, s_blk)][None, :]
        csz = cs_ref[pl.ds(2 * S + s * s_blk, s_blk)][None, :]
        cs2 = csx * csx + csy * csy + csz * csz
        shx = sh_ref[0, :, 0:1]
        shy = sh_ref[0, :, 1:2]
        shz = sh_ref[0, :, 2:3]
        sh2 = shx * shx + shy * shy + shz * shz
        lhs = jnp.concatenate(
            [-2.0 * shx, -2.0 * shy, -2.0 * shz, jnp.ones_like(shx), sh2],
            axis=1,
        )
        rhs = jnp.concatenate(
            [csx, csy, csz, cs2, jnp.ones_like(csx)], axis=0
        )
        lh = lhs.astype(jnp.bfloat16)
        ll = (lhs - lh.astype(jnp.float32)).astype(jnp.bfloat16)
        rh = rhs.astype(jnp.bfloat16)
        rl = (rhs - rh.astype(jnp.float32)).astype(jnp.bfloat16)
        d = jnp.dot(
            jnp.concatenate([lh, lh, ll], axis=1),
            jnp.concatenate([rh, rl, rh], axis=0),
            preferred_element_type=jnp.float32,
        )

        c2s = jnp.min(d, axis=0)
        c2s = jnp.where(c2s > LAMBD, c2s, 0.0)
        acc_ref[0, 0] += jnp.sum(c2s)
        acc_ref[1, 0] = jnp.maximum(acc_ref[1, 0], jnp.max(c2s))
        m = d[:, 0:128]
        for i in range(1, s_blk // 128):
            m = jnp.minimum(m, d[:, i * 128 : (i + 1) * 128])
        s2c_ref[...] = jnp.minimum(s2c_ref[...], m)

        @pl.when(s == nsb - 1)
        def _finalize():
            s2c = jnp.min(s2c_ref[...], axis=1, keepdims=True)
            s2c = jnp.where(s2c > LAMBD, s2c, 0.0)
            loss_b = (
                jnp.mean(s2c) * gamma_eff
                + acc_ref[0, 0] / S
                + BETA * acc_ref[1, 0]
            )

            @pl.when(b == 0)
            def _():
                loss_ref[0, 0] = loss_b / B

            @pl.when(b != 0)
            def _():
                loss_ref[0, 0] += loss_b / B

    return pl.pallas_call(
        body,
        grid=(B, nsb),
        in_specs=[
            pl.BlockSpec((3 * S,), lambda b, s: (b,)),
            pl.BlockSpec((1, N, 3), lambda b, s: (b, 0, 0)),
        ],
        out_specs=pl.BlockSpec(memory_space=pltpu.SMEM),
        out_shape=jax.ShapeDtypeStruct((1, 1), jnp.float32),
        scratch_shapes=[
            pltpu.VMEM((N, 128), jnp.float32),
            pltpu.SMEM((2, 1), jnp.float32),
        ],
    )


def kernel(cage_v, cage_f, shape):
    B, M, _ = cage_v.shape
    F = cage_f.shape[1]
    N = shape.shape[1]
    S = F * _K

    cv_flat = cage_v.reshape(-1)
    cf_flat = cage_f.reshape(-1)

    cs = _make_sampler(B, M, F)(cv_flat, cf_flat)
    loss = _make_chamfer(B, M, S, N, 5120)(cs, shape)
    return loss[0, 0]
